# baseline (speedup 1.0000x reference)
import functools

import jax
import jax.numpy as jnp
from jax import lax
from jax.experimental import pallas as pl
from jax.experimental.pallas import tpu as pltpu
from jax.experimental.pallas import tpu_sc as plsc

VOCAB = 1000000
EMBED = 64
BATCH = 16384

_CHUNK = 128


def _build_kernel():
  info = plsc.get_sparse_core_info()
  nc, ns = info.num_cores, info.num_subcores
  nw = nc * ns
  b_per_w = BATCH // nw
  n_chunks = b_per_w // _CHUNK
  rows_per_w = n_chunks

  mesh = plsc.VectorSubcoreMesh(core_axis_name="c", subcore_axis_name="s")

  @functools.partial(
      pl.kernel,
      mesh=mesh,
      compiler_params=pltpu.CompilerParams(use_tc_tiling_on_sc=False),
      out_type=(
          jax.ShapeDtypeStruct((BATCH, EMBED), jnp.float32),
          jax.ShapeDtypeStruct((BATCH, EMBED), jnp.float32),
      ),
      scratch_types=[
          pltpu.VMEM((rows_per_w, _CHUNK), jnp.int32),
          pltpu.VMEM((rows_per_w, _CHUNK), jnp.int32),
          pltpu.VMEM((b_per_w, EMBED), jnp.float32),
          pltpu.VMEM((b_per_w, EMBED), jnp.float32),
          pltpu.SemaphoreType.DMA,
          pltpu.SemaphoreType.DMA,
      ],
  )
  def lookup(center_hbm, context_hbm, word_hbm, ctx_hbm,
             out_c, out_x, cidx_v, xidx_v, crows_v, xrows_v, sem_c, sem_x):
    wid = lax.axis_index("s") * nc + lax.axis_index("c")
    row0 = wid * rows_per_w
    base = wid * b_per_w

    pltpu.sync_copy(center_hbm.at[pl.ds(row0, rows_per_w)], cidx_v)
    pltpu.sync_copy(context_hbm.at[pl.ds(row0, rows_per_w)], xidx_v)

    copies = []
    for j in range(n_chunks):
      copies.append(pltpu.async_copy(
          word_hbm.at[cidx_v.at[j]],
          crows_v.at[pl.ds(j * _CHUNK, _CHUNK)], sem_c))
      copies.append(pltpu.async_copy(
          ctx_hbm.at[xidx_v.at[j]],
          xrows_v.at[pl.ds(j * _CHUNK, _CHUNK)], sem_x))
    for c in copies:
      c.wait()

    pltpu.sync_copy(crows_v, out_c.at[pl.ds(base, b_per_w)])
    pltpu.sync_copy(xrows_v, out_x.at[pl.ds(base, b_per_w)])

  return lookup


_lookup = _build_kernel()


@jax.jit
def kernel(center, context, word_embeds, context_embeds):
  c2 = center.astype(jnp.int32).reshape(-1, _CHUNK)
  x2 = context.astype(jnp.int32).reshape(-1, _CHUNK)
  return _lookup(c2, x2, word_embeds, context_embeds)


# speedup vs baseline: 1.5815x; 1.5815x over previous
import functools

import jax
import jax.numpy as jnp
from jax import lax
from jax._src import core as _jax_core
from jax._src.pallas import core as _pallas_core
from jax.experimental import pallas as pl
from jax.experimental.pallas import tpu as pltpu
from jax.experimental.pallas import tpu_sc as plsc


def _to_default_space(x):
  return _pallas_core.with_memory_space_constraint_p.bind(
      x, memory_space=_jax_core.MemorySpace.Device)

VOCAB = 1000000
EMBED = 64
BATCH = 16384

_HALF = 256


def _build_kernel():
  info = plsc.get_sparse_core_info()
  nc, ns = info.num_cores, info.num_subcores
  nw = nc * ns
  b_per_w = BATCH // nw
  n_halves = b_per_w // _HALF

  mesh = plsc.VectorSubcoreMesh(core_axis_name="c", subcore_axis_name="s")

  @functools.partial(
      pl.kernel,
      mesh=mesh,
      out_type=(
          pltpu.HBM((BATCH, EMBED), jnp.float32),
          pltpu.HBM((BATCH, EMBED), jnp.float32),
      ),
      scratch_types=[
          pltpu.VMEM((b_per_w,), jnp.int32),
          pltpu.VMEM((b_per_w,), jnp.int32),
          pltpu.VMEM((_HALF, EMBED), jnp.float32),
          pltpu.VMEM((_HALF, EMBED), jnp.float32),
          pltpu.SemaphoreType.DMA,
          pltpu.SemaphoreType.DMA,
      ],
  )
  def lookup(center_hbm, context_hbm, word_hbm, ctx_hbm,
             out_c, out_x, cidx_v, xidx_v, crows_v, xrows_v, sem_c, sem_x):
    wid = lax.axis_index("s") * nc + lax.axis_index("c")
    base = wid * b_per_w

    pltpu.sync_copy(center_hbm.at[pl.ds(base, b_per_w)], cidx_v)
    pltpu.sync_copy(context_hbm.at[pl.ds(base, b_per_w)], xidx_v)

    for half in range(n_halves):
      def group_body(g, _):
        cv = cidx_v[pl.ds(half * _HALF + g * 16, 16)]
        xv = xidx_v[pl.ds(half * _HALF + g * 16, 16)]
        for lane in range(16):
          pltpu.async_copy(word_hbm.at[pl.ds(cv[lane], 1)],
                           crows_v.at[pl.ds(g * 16 + lane, 1)], sem_c)
          pltpu.async_copy(ctx_hbm.at[pl.ds(xv[lane], 1)],
                           xrows_v.at[pl.ds(g * 16 + lane, 1)], sem_x)
        return 0

      lax.fori_loop(0, _HALF // 16, group_body, 0)

      pltpu.make_async_copy(word_hbm.at[pl.ds(0, _HALF)], crows_v,
                            sem_c).wait()
      pltpu.make_async_copy(ctx_hbm.at[pl.ds(0, _HALF)], xrows_v,
                            sem_x).wait()

      pltpu.sync_copy(crows_v, out_c.at[pl.ds(base + half * _HALF, _HALF)])
      pltpu.sync_copy(xrows_v, out_x.at[pl.ds(base + half * _HALF, _HALF)])

  return lookup


_lookup = _build_kernel()


@jax.jit
def kernel(center, context, word_embeds, context_embeds):
  out_c, out_x = _lookup(center.astype(jnp.int32), context.astype(jnp.int32),
                         word_embeds, context_embeds)
  return _to_default_space(out_c), _to_default_space(out_x)
